# baseline (speedup 1.0000x reference)
import functools

import jax
import jax.numpy as jnp
from jax import lax
from jax.experimental import pallas as pl
from jax.experimental.pallas import tpu as pltpu
from jax.experimental.pallas import tpu_sc as plsc

N = 10000
E = 320000
D = 128
LAYERS = 3

NC = 2
NS = 16
NW = NC * NS
K = 128
NPHASE = 2
HCHUNK = 40
PER_W = K * HCHUNK * NPHASE
E_PAD = NW * PER_W
ROWS_MAIN = 624
ROWS_REM = N - ROWS_MAIN * NS

_mesh = plsc.VectorSubcoreMesh(core_axis_name="c", subcore_axis_name="s")


@functools.partial(
    pl.kernel,
    out_type=jax.ShapeDtypeStruct((NC, N, D), jnp.float32),
    mesh=_mesh,
    scratch_types=[
        pltpu.VMEM((HCHUNK, K), jnp.int32),
        pltpu.VMEM((HCHUNK, K), jnp.int32),
        pltpu.VMEM((HCHUNK, K), jnp.float32),
        pltpu.VMEM((K, D), jnp.float32),
        pltpu.VMEM((K, D), jnp.float32),
        pltpu.VMEM_SHARED((N, D), jnp.float32),
        pltpu.SemaphoreType.DMA,
        pltpu.SemaphoreType.DMA,
    ],
)
def _spmm_sc(ebs_hbm, col_hbm, row_hbm, w_hbm, zeros_hbm, out_hbm,
             col_v, row_v, w_v, rows0, rows1, acc, gsem0, gsem1):
    c = lax.axis_index("c")
    s = lax.axis_index("s")
    wid = c * NS + s
    rows = (rows0, rows1)
    gsems = (gsem0, gsem1)

    base = s * ROWS_MAIN
    pltpu.sync_copy(zeros_hbm.at[pl.ds(base, ROWS_MAIN)],
                    acc.at[pl.ds(base, ROWS_MAIN)])

    @pl.when(s == 0)
    def _zero_rem():
        pltpu.sync_copy(zeros_hbm.at[pl.ds(ROWS_MAIN * NS, ROWS_REM)],
                        acc.at[pl.ds(ROWS_MAIN * NS, ROWS_REM)])

    plsc.subcore_barrier()

    for phase in range(NPHASE):
        pltpu.sync_copy(col_hbm.at[wid, phase], col_v)
        pltpu.sync_copy(row_hbm.at[wid, phase], row_v)
        pltpu.sync_copy(w_hbm.at[wid, phase], w_v)

        for b in range(2):
            pltpu.async_copy(ebs_hbm.at[col_v.at[b]], rows[b], gsems[b])

        @pl.loop(0, HCHUNK, step=2)
        def chunk_body(g):
            for b in range(2):
                gg = g + b
                rbuf = rows[b]
                pltpu.make_async_copy(ebs_hbm.at[col_v.at[gg]], rbuf,
                                      gsems[b]).wait()

                def scale_body(ib, carry2):
                    wvec = w_v[gg, pl.ds(ib * 16, 16)]
                    for j in range(16):
                        w_s = wvec[j]
                        i = ib * 16 + j
                        for jj in range(D // 16):
                            sl = pl.ds(jj * 16, 16)
                            rbuf[i, sl] = rbuf[i, sl] * w_s
                    return carry2

                lax.fori_loop(0, K // 16, scale_body, 0)
                pltpu.sync_copy(rbuf, acc.at[row_v.at[gg]], add=True)

                @pl.when(gg + 2 < HCHUNK)
                def _prefetch():
                    pltpu.async_copy(ebs_hbm.at[col_v.at[gg + 2]], rbuf,
                                     gsems[b])

    plsc.subcore_barrier()
    pltpu.sync_copy(acc.at[pl.ds(base, ROWS_MAIN)],
                    out_hbm.at[c, pl.ds(base, ROWS_MAIN)])

    @pl.when(s == 0)
    def _out_rem():
        pltpu.sync_copy(acc.at[pl.ds(ROWS_MAIN * NS, ROWS_REM)],
                        out_hbm.at[c, pl.ds(ROWS_MAIN * NS, ROWS_REM)])


def _dense_body(p_ref, old_ref, ws_ref, wd_ref, out_ref):
    l_side = p_ref[0] + p_ref[1]
    old = old_ref[...]
    li = l_side + old
    acc = jnp.dot(li, ws_ref[...], preferred_element_type=jnp.float32)
    acc = acc + jnp.dot(l_side * old, wd_ref[...],
                        preferred_element_type=jnp.float32)
    out_ref[...] = jnp.where(acc >= 0, acc, 0.2 * acc)


def _dense(parts, old, ws, wd):
    R = 2000
    return pl.pallas_call(
        _dense_body,
        grid=(N // R,),
        in_specs=[
            pl.BlockSpec((2, R, D), lambda i: (0, i, 0)),
            pl.BlockSpec((R, D), lambda i: (i, 0)),
            pl.BlockSpec((D, D), lambda i: (0, 0)),
            pl.BlockSpec((D, D), lambda i: (0, 0)),
        ],
        out_specs=pl.BlockSpec((R, D), lambda i: (i, 0)),
        out_shape=jax.ShapeDtypeStruct((N, D), jnp.float32),
    )(parts, old, ws, wd)


def kernel(initial_ebs, edge_index, edge_weight, W_sides, W_dots):
    row = edge_index[0].astype(jnp.int32)
    col = edge_index[1].astype(jnp.int32)
    w = edge_weight.astype(jnp.float32)
    pad = E_PAD - E
    shp = (NW, NPHASE, HCHUNK, K)
    colp = jnp.concatenate([col, jnp.zeros((pad,), jnp.int32)]).reshape(shp)
    rowp = jnp.concatenate([row, jnp.zeros((pad,), jnp.int32)]).reshape(shp)
    wp = jnp.concatenate([w, jnp.zeros((pad,), jnp.float32)]).reshape(shp)
    zeros = jnp.zeros((N, D), jnp.float32)

    old = initial_ebs
    outs = []
    for layer_no in range(LAYERS):
        parts = _spmm_sc(old, colp, rowp, wp, zeros)
        fold = _dense(parts, old, W_sides[layer_no], W_dots[layer_no])
        outs.append(fold)
        old = fold
    return jnp.concatenate(outs, axis=0)


# speedup vs baseline: 1.2182x; 1.2182x over previous
import functools

import jax
import jax.numpy as jnp
from jax import lax
from jax.experimental import pallas as pl
from jax.experimental.pallas import tpu as pltpu
from jax.experimental.pallas import tpu_sc as plsc

N = 10000
E = 320000
D = 128
LAYERS = 3

NC = 2
NS = 16
NW = NC * NS
K = 128
NCHUNK = 79
PER_W = K * NCHUNK
E_PAD = NW * PER_W
ROWS_MAIN = 624
ROWS_REM = N - ROWS_MAIN * NS

_mesh = plsc.VectorSubcoreMesh(core_axis_name="c", subcore_axis_name="s")


@functools.partial(
    pl.kernel,
    out_type=jax.ShapeDtypeStruct((NC, N, D), jnp.float32),
    mesh=_mesh,
    scratch_types=[
        pltpu.VMEM((NCHUNK, K), jnp.int32),
        pltpu.VMEM((NCHUNK, K), jnp.int32),
        pltpu.VMEM((NCHUNK, K), jnp.float32),
        pltpu.VMEM((K, D), jnp.float32),
        pltpu.VMEM_SHARED((N, D), jnp.float32),
        pltpu.SemaphoreType.DMA,
        pltpu.SemaphoreType.DMA,
        pltpu.SemaphoreType.DMA,
        pltpu.SemaphoreType.DMA,
    ],
)
def _spmm_sc(ebs_hbm, col_hbm, row_hbm, w_hbm, zeros_hbm, out_hbm,
             col_v, row_v, w_v, rows_v, acc, sem0, sem1, sem2, sem3):
    sems = (sem0, sem1, sem2, sem3)
    c = lax.axis_index("c")
    s = lax.axis_index("s")
    wid = c * NS + s

    base = s * ROWS_MAIN
    pltpu.sync_copy(zeros_hbm.at[pl.ds(base, ROWS_MAIN)],
                    acc.at[pl.ds(base, ROWS_MAIN)])

    @pl.when(s == 0)
    def _zero_rem():
        pltpu.sync_copy(zeros_hbm.at[pl.ds(ROWS_MAIN * NS, ROWS_REM)],
                        acc.at[pl.ds(ROWS_MAIN * NS, ROWS_REM)])
    pltpu.sync_copy(col_hbm.at[wid], col_v)
    pltpu.sync_copy(row_hbm.at[wid], row_v)
    pltpu.sync_copy(w_hbm.at[wid], w_v)
    plsc.subcore_barrier()

    QK = K // 4

    def chunk_body(g, carry):
        for q in range(4):
            pltpu.async_copy(ebs_hbm.at[col_v.at[g, pl.ds(q * QK, QK)]],
                             rows_v.at[pl.ds(q * QK, QK)], sems[q])
        for q in range(4):
            pltpu.make_async_copy(ebs_hbm.at[col_v.at[g, pl.ds(q * QK, QK)]],
                                  rows_v.at[pl.ds(q * QK, QK)],
                                  sems[q]).wait()

        def scale_body(ib, carry2):
            wvec = w_v[g, pl.ds(ib * 16, 16)]
            for j in range(16):
                w_s = wvec[j]
                i = ib * 16 + j
                for jj in range(D // 16):
                    sl = pl.ds(jj * 16, 16)
                    rows_v[i, sl] = rows_v[i, sl] * w_s
            return carry2

        lax.fori_loop(0, K // 16, scale_body, 0)
        pltpu.sync_copy(rows_v, acc.at[row_v.at[g]], add=True)
        return carry

    lax.fori_loop(0, NCHUNK, chunk_body, 0)
    plsc.subcore_barrier()
    pltpu.sync_copy(acc.at[pl.ds(base, ROWS_MAIN)],
                    out_hbm.at[c, pl.ds(base, ROWS_MAIN)])

    @pl.when(s == 0)
    def _out_rem():
        pltpu.sync_copy(acc.at[pl.ds(ROWS_MAIN * NS, ROWS_REM)],
                        out_hbm.at[c, pl.ds(ROWS_MAIN * NS, ROWS_REM)])


def _dense_body(p_ref, old_ref, ws_ref, wd_ref, out_ref):
    l_side = p_ref[0] + p_ref[1]
    old = old_ref[...]
    li = l_side + old
    acc = jnp.dot(li, ws_ref[...], preferred_element_type=jnp.float32)
    acc = acc + jnp.dot(l_side * old, wd_ref[...],
                        preferred_element_type=jnp.float32)
    out_ref[...] = jnp.where(acc >= 0, acc, 0.2 * acc)


def _dense(parts, old, ws, wd):
    R = 2000
    return pl.pallas_call(
        _dense_body,
        grid=(N // R,),
        in_specs=[
            pl.BlockSpec((2, R, D), lambda i: (0, i, 0)),
            pl.BlockSpec((R, D), lambda i: (i, 0)),
            pl.BlockSpec((D, D), lambda i: (0, 0)),
            pl.BlockSpec((D, D), lambda i: (0, 0)),
        ],
        out_specs=pl.BlockSpec((R, D), lambda i: (i, 0)),
        out_shape=jax.ShapeDtypeStruct((N, D), jnp.float32),
    )(parts, old, ws, wd)


def kernel(initial_ebs, edge_index, edge_weight, W_sides, W_dots):
    row = edge_index[0].astype(jnp.int32)
    col = edge_index[1].astype(jnp.int32)
    w = edge_weight.astype(jnp.float32)
    pad = E_PAD - E
    colp = jnp.concatenate([col, jnp.zeros((pad,), jnp.int32)]).reshape(
        NW, NCHUNK, K)
    rowp = jnp.concatenate([row, jnp.zeros((pad,), jnp.int32)]).reshape(
        NW, NCHUNK, K)
    wp = jnp.concatenate([w, jnp.zeros((pad,), jnp.float32)]).reshape(
        NW, NCHUNK, K)
    zeros = jnp.zeros((N, D), jnp.float32)

    old = initial_ebs
    outs = []
    for layer_no in range(LAYERS):
        parts = _spmm_sc(old, colp, rowp, wp, zeros)
        fold = _dense(parts, old, W_sides[layer_no], W_dots[layer_no])
        outs.append(fold)
        old = fold
    return jnp.concatenate(outs, axis=0)
